# baseline (speedup 1.0000x reference)
import jax
import jax.numpy as jnp
from jax.experimental import pallas as pl
from jax.experimental.pallas import tpu as pltpu

ALPHA = 0.25
GAMMA = 2.0

_ROWS = 2048


def _focal_body(pred_ref, true_ref, loss_ref, npos_ref):
    x = pred_ref[...]
    t = true_ref[...]
    union = (x != 0.0) | (t != 0.0)
    p = jax.nn.sigmoid(x)
    ce = jnp.maximum(x, 0.0) - x * t + jnp.log1p(jnp.exp(-jnp.abs(x)))
    p_t = p * t + (1.0 - p) * (1.0 - t)
    loss = ce * (1.0 - p_t) ** 2
    alpha_t = ALPHA * t + (1.0 - ALPHA) * (1.0 - t)
    val = jnp.where(union, alpha_t * loss, 0.0)
    part_loss = jnp.sum(val)
    part_npos = jnp.sum((t > 0.5).astype(jnp.float32))

    @pl.when(pl.program_id(0) == 0)
    def _init():
        loss_ref[0] = 0.0
        npos_ref[0] = 0.0

    loss_ref[0] += part_loss
    npos_ref[0] += part_npos


def kernel(predicted_foreground_masks, peak_normalized_images):
    pred = predicted_foreground_masks.reshape(-1, 512)
    true = peak_normalized_images.reshape(-1, 512)
    n_rows = pred.shape[0]
    grid = n_rows // _ROWS

    loss_sum, npos = pl.pallas_call(
        _focal_body,
        grid=(grid,),
        in_specs=[
            pl.BlockSpec((_ROWS, 512), lambda i: (i, 0)),
            pl.BlockSpec((_ROWS, 512), lambda i: (i, 0)),
        ],
        out_specs=[
            pl.BlockSpec(memory_space=pltpu.SMEM),
            pl.BlockSpec(memory_space=pltpu.SMEM),
        ],
        out_shape=[
            jax.ShapeDtypeStruct((1,), jnp.float32),
            jax.ShapeDtypeStruct((1,), jnp.float32),
        ],
    )(pred, true)

    return loss_sum[0] / jnp.maximum(npos[0], 1.0)


# speedup vs baseline: 1.5011x; 1.5011x over previous
import jax
import jax.numpy as jnp
from jax.experimental import pallas as pl
from jax.experimental.pallas import tpu as pltpu

ALPHA = 0.25
GAMMA = 2.0

_ROWS = 2048
_CH = 16


def _focal_body(pred_ref, true_ref, loss_ref, npos_ref):
    def step(i, carry):
        acc_l, acc_n = carry
        x = pred_ref[pl.ds(i * _CH, _CH), :]
        t = true_ref[pl.ds(i * _CH, _CH), :]
        ax = jnp.abs(x)
        e = jnp.exp(-ax)
        one_pe = 1.0 + e
        sp = jnp.log1p(e)
        ce = jnp.maximum(x, 0.0) - x * t + sp
        r = 1.0 / one_pe
        p = jnp.where(x >= 0.0, r, e * r)
        q = p + t - 2.0 * (p * t)
        at = 0.75 - 0.5 * t
        val = ce * (q * q) * at
        val = jnp.where(ax + t != 0.0, val, 0.0)
        acc_l = acc_l + val
        acc_n = acc_n + jnp.where(t > 0.5, 1.0, 0.0)
        return acc_l, acc_n

    z = jnp.zeros((_CH, 512), jnp.float32)
    acc_l, acc_n = jax.lax.fori_loop(0, _ROWS // _CH, step, (z, z))
    part_loss = jnp.sum(acc_l)
    part_npos = jnp.sum(acc_n)

    @pl.when(pl.program_id(0) == 0)
    def _init():
        loss_ref[0] = 0.0
        npos_ref[0] = 0.0

    loss_ref[0] += part_loss
    npos_ref[0] += part_npos


def kernel(predicted_foreground_masks, peak_normalized_images):
    pred = predicted_foreground_masks.reshape(-1, 512)
    true = peak_normalized_images.reshape(-1, 512)
    n_rows = pred.shape[0]
    grid = n_rows // _ROWS

    loss_sum, npos = pl.pallas_call(
        _focal_body,
        grid=(grid,),
        in_specs=[
            pl.BlockSpec((_ROWS, 512), lambda i: (i, 0)),
            pl.BlockSpec((_ROWS, 512), lambda i: (i, 0)),
        ],
        out_specs=[
            pl.BlockSpec(memory_space=pltpu.SMEM),
            pl.BlockSpec(memory_space=pltpu.SMEM),
        ],
        out_shape=[
            jax.ShapeDtypeStruct((1,), jnp.float32),
            jax.ShapeDtypeStruct((1,), jnp.float32),
        ],
    )(pred, true)

    return loss_sum[0] / jnp.maximum(npos[0], 1.0)
